# baseline (speedup 1.0000x reference)
import functools

import jax
import jax.numpy as jnp
from jax import lax
from jax.experimental import pallas as pl
from jax.experimental.pallas import tpu as pltpu
from jax.experimental.pallas import tpu_sc as plsc

_N = 10000
_E = 320000
_D = 128
_NP = 10112
_RPT = _NP // 16
_CK = 128
_CH = 80
_TILES = 32
_EPAD = _TILES * _CH * _CK
_NB = 2
_QC = 40


def _sc_mesh():
    return plsc.VectorSubcoreMesh(
        core_axis_name="c", subcore_axis_name="s", num_cores=2, num_subcores=16
    )


def _deg_body(dst_hbm, init_hbm, ones_hbm, out_hbm, dst_v, ones_v, acc_sh):
    c = lax.axis_index("c")
    s = lax.axis_index("s")
    wid = c * 16 + s
    pltpu.sync_copy(dst_hbm.at[wid], dst_v)
    pltpu.sync_copy(ones_hbm, ones_v)
    pltpu.sync_copy(init_hbm.at[pl.ds(s * _RPT, _RPT)],
                    acc_sh.at[pl.ds(s * _RPT, _RPT)])
    plsc.subcore_barrier()

    @pl.loop(0, _CH)
    def _chunk(j):
        pltpu.sync_copy(ones_v, acc_sh.at[dst_v.at[j]], add=True)

    plsc.subcore_barrier()
    pltpu.sync_copy(acc_sh.at[pl.ds(s * _RPT, _RPT)],
                    out_hbm.at[c, pl.ds(s * _RPT, _RPT)])


def _deg_call(dst_idx):
    k = pl.kernel(
        _deg_body,
        out_type=jax.ShapeDtypeStruct((2, _NP, 8), jnp.float32),
        mesh=_sc_mesh(),
        scratch_types=[
            pltpu.VMEM((_CH, _CK), jnp.int32),
            pltpu.VMEM((_CK, 8), jnp.float32),
            pltpu.VMEM_SHARED((_NP, 8), jnp.float32),
        ],
    )
    init = jnp.full((_NP, 8), 0.5, dtype=jnp.float32)
    ones = jnp.ones((_CK, 8), dtype=jnp.float32)
    parts = k(dst_idx, init, ones)
    return parts[0] + parts[1]


def _spmm_body(z_hbm, src_hbm, dst_hbm, zeros_hbm, out_hbm,
               src_v, dst_v,
               r0, r1,
               g0, g1, acc_sh):
    c = lax.axis_index("c")
    s = lax.axis_index("s")
    wid = c * 16 + s
    rows = (r0, r1)
    gsem = (g0, g1)
    pltpu.sync_copy(zeros_hbm, acc_sh.at[pl.ds(s * _RPT, _RPT)])
    plsc.subcore_barrier()

    for q in range(_CH // _QC):
        pltpu.sync_copy(src_hbm.at[wid, pl.ds(q * _QC, _QC)], src_v)
        pltpu.sync_copy(dst_hbm.at[wid, pl.ds(q * _QC, _QC)], dst_v)
        for b in range(_NB):
            pltpu.async_copy(z_hbm.at[src_v.at[b]], rows[b], gsem[b])

        @pl.loop(0, _QC, step=_NB)
        def _chunks(j):
            for b in range(_NB):
                jb = j + b
                pltpu.make_async_copy(
                    z_hbm.at[src_v.at[jb]], rows[b], gsem[b]).wait()
                pltpu.sync_copy(rows[b], acc_sh.at[dst_v.at[jb]], add=True)

                @pl.when(jb + _NB < _QC)
                def _():
                    pltpu.async_copy(
                        z_hbm.at[src_v.at[jb + _NB]], rows[b], gsem[b])

    plsc.subcore_barrier()
    pltpu.sync_copy(acc_sh.at[pl.ds(s * _RPT, _RPT)],
                    out_hbm.at[c, pl.ds(s * _RPT, _RPT)])


def _spmm_call(z, src_idx, dst_idx, zeros_rpt):
    k = pl.kernel(
        _spmm_body,
        out_type=jax.ShapeDtypeStruct((2, _NP, _D), jnp.float32),
        mesh=_sc_mesh(),
        scratch_types=[
            pltpu.VMEM((_QC, _CK), jnp.int32),
            pltpu.VMEM((_QC, _CK), jnp.int32),
            pltpu.VMEM((_CK, _D), jnp.float32),
            pltpu.VMEM((_CK, _D), jnp.float32),
            pltpu.SemaphoreType.DMA,
            pltpu.SemaphoreType.DMA,
            pltpu.VMEM_SHARED((_NP, _D), jnp.float32),
        ],
    )
    return k(z, src_idx, dst_idx, zeros_rpt)


_R = 128
_G = _NP // _R


def _dis(deg_ref):
    return lax.rsqrt(deg_ref[:, 0:1])


def _mm_first_body(x_ref, w_ref, deg_ref, o_ref):
    o_ref[...] = (
        jnp.dot(x_ref[...], w_ref[...], preferred_element_type=jnp.float32)
        * _dis(deg_ref)
    )


def _mm_first(x, w, deg8):
    return pl.pallas_call(
        _mm_first_body,
        grid=(_G,),
        in_specs=[
            pl.BlockSpec((_R, _D), lambda i: (i, 0)),
            pl.BlockSpec((_D, _D), lambda i: (0, 0)),
            pl.BlockSpec((_R, 8), lambda i: (i, 0)),
        ],
        out_specs=pl.BlockSpec((_R, _D), lambda i: (i, 0)),
        out_shape=jax.ShapeDtypeStruct((_NP, _D), jnp.float32),
    )(x, w, deg8)


def _mm_mid_body(acc_ref, z_ref, deg_ref, b_ref, w_ref, o_ref):
    dis = _dis(deg_ref)
    h = jnp.maximum((acc_ref[0] + acc_ref[1] + z_ref[...]) * dis + b_ref[...], 0.0)
    o_ref[...] = jnp.dot(h, w_ref[...], preferred_element_type=jnp.float32) * dis


def _mm_mid(acc, z, deg8, b, w):
    return pl.pallas_call(
        _mm_mid_body,
        grid=(_G,),
        in_specs=[
            pl.BlockSpec((2, _R, _D), lambda i: (0, i, 0)),
            pl.BlockSpec((_R, _D), lambda i: (i, 0)),
            pl.BlockSpec((_R, 8), lambda i: (i, 0)),
            pl.BlockSpec((1, _D), lambda i: (0, 0)),
            pl.BlockSpec((_D, _D), lambda i: (0, 0)),
        ],
        out_specs=pl.BlockSpec((_R, _D), lambda i: (i, 0)),
        out_shape=jax.ShapeDtypeStruct((_NP, _D), jnp.float32),
    )(acc, z, deg8, b, w)


def _final_body(acc_ref, z_ref, deg_ref, b_ref, o_ref):
    o_ref[...] = (
        (acc_ref[0] + acc_ref[1] + z_ref[...]) * _dis(deg_ref) + b_ref[...]
    )


def _final(acc, z, deg8, b):
    return pl.pallas_call(
        _final_body,
        grid=(_G,),
        in_specs=[
            pl.BlockSpec((2, _R, _D), lambda i: (0, i, 0)),
            pl.BlockSpec((_R, _D), lambda i: (i, 0)),
            pl.BlockSpec((_R, 8), lambda i: (i, 0)),
            pl.BlockSpec((1, _D), lambda i: (0, 0)),
        ],
        out_specs=pl.BlockSpec((_R, _D), lambda i: (i, 0)),
        out_shape=jax.ShapeDtypeStruct((_NP, _D), jnp.float32),
    )(acc, z, deg8, b)


def _prep_edges(edge_index):
    pad = jnp.full((_EPAD - _E,), _N, dtype=jnp.int32)
    src = jnp.concatenate([edge_index[0], pad]).reshape(_TILES, _CH, _CK)
    dst = jnp.concatenate([edge_index[1], pad]).reshape(_TILES, _CH, _CK)
    return src, dst


def _branch(x, edge_index, params, zeros_rpt):
    w1, b1, w2, b2, w3, b3, w4, b4 = params
    src, dst = _prep_edges(edge_index)
    xp = jnp.pad(x, ((0, _NP - _N), (0, 0)))
    deg8 = _deg_call(dst)
    z = _mm_first(xp, w1, deg8)
    ws = (w2, w3, w4)
    bs = (b1.reshape(1, _D), b2.reshape(1, _D), b3.reshape(1, _D),
          b4.reshape(1, _D))
    for layer in range(3):
        acc = _spmm_call(z, src, dst, zeros_rpt)
        z = _mm_mid(acc, z, deg8, bs[layer], ws[layer])
    acc = _spmm_call(z, src, dst, zeros_rpt)
    out = _final(acc, z, deg8, bs[3])
    return out[:_N]


def kernel(x_data_matrix, x_edge_index, y_data_matrix, y_edge_index,
           Wx1, bx1, Wx2, bx2, Wx3, bx3, Wx4, bx4,
           Wy1, by1, Wy2, by2, Wy3, by3, Wy4, by4):
    zeros_rpt = jnp.zeros((_RPT, _D), dtype=jnp.float32)
    xo = _branch(x_data_matrix, x_edge_index,
                 (Wx1, bx1, Wx2, bx2, Wx3, bx3, Wx4, bx4), zeros_rpt)
    xo, y_ei = lax.optimization_barrier((xo, y_edge_index))
    yo = _branch(y_data_matrix, y_ei,
                 (Wy1, by1, Wy2, by2, Wy3, by3, Wy4, by4), zeros_rpt)
    return (xo, yo)


# speedup vs baseline: 1.4384x; 1.4384x over previous
import jax
import jax.numpy as jnp
from jax import lax
from jax.experimental import pallas as pl
from jax.experimental.pallas import tpu as pltpu
from jax.experimental.pallas import tpu_sc as plsc

_N = 10000
_E = 320000
_D = 128
_NP = 10112
_RPT = _NP // 16
_CK = 128
_CH = 160
_EPAD = 16 * _CH * _CK
_NB = 2
_QC = 40


def _sc_mesh():
    return plsc.VectorSubcoreMesh(
        core_axis_name="c", subcore_axis_name="s", num_cores=2, num_subcores=16
    )


def _spmm_body(z_hbm, src_hbm, dst_hbm, zeros_hbm, out_hbm,
               src_v, dst_v, r0, r1, g0, g1, acc_sh):
    c = lax.axis_index("c")
    s = lax.axis_index("s")
    rows = (r0, r1)
    gsem = (g0, g1)
    pltpu.sync_copy(zeros_hbm, acc_sh.at[pl.ds(s * _RPT, _RPT)])
    plsc.subcore_barrier()

    for q in range(_CH // _QC):
        pltpu.sync_copy(src_hbm.at[c, s, pl.ds(q * _QC, _QC)], src_v)
        pltpu.sync_copy(dst_hbm.at[c, s, pl.ds(q * _QC, _QC)], dst_v)
        for b in range(_NB):
            pltpu.async_copy(z_hbm.at[c].at[src_v.at[b]], rows[b], gsem[b])

        @pl.loop(0, _QC, step=_NB)
        def _chunks(j):
            for b in range(_NB):
                jb = j + b
                pltpu.make_async_copy(
                    z_hbm.at[c].at[src_v.at[jb]], rows[b], gsem[b]).wait()
                pltpu.sync_copy(rows[b], acc_sh.at[dst_v.at[jb]], add=True)

                @pl.when(jb + _NB < _QC)
                def _():
                    pltpu.async_copy(
                        z_hbm.at[c].at[src_v.at[jb + _NB]], rows[b], gsem[b])

    plsc.subcore_barrier()
    pltpu.sync_copy(acc_sh.at[pl.ds(s * _RPT, _RPT)],
                    out_hbm.at[c, pl.ds(s * _RPT, _RPT)])


def _spmm_call(z2, src_idx, dst_idx, zeros_rpt):
    k = pl.kernel(
        _spmm_body,
        out_type=jax.ShapeDtypeStruct((2, _NP, _D), jnp.float32),
        mesh=_sc_mesh(),
        scratch_types=[
            pltpu.VMEM((_QC, _CK), jnp.int32),
            pltpu.VMEM((_QC, _CK), jnp.int32),
            pltpu.VMEM((_CK, _D), jnp.float32),
            pltpu.VMEM((_CK, _D), jnp.float32),
            pltpu.SemaphoreType.DMA,
            pltpu.SemaphoreType.DMA,
            pltpu.VMEM_SHARED((_NP, _D), jnp.float32),
        ],
    )
    return k(z2, src_idx, dst_idx, zeros_rpt)


_R = 128
_G = _NP // _R


def _dis(deg_ref):
    return lax.rsqrt(1.0 + deg_ref[0, :, 0:1])


def _mm_first_body(x_ref, w_ref, deg_ref, o_ref):
    o_ref[0] = (
        jnp.dot(x_ref[0], w_ref[0], preferred_element_type=jnp.float32)
        * _dis(deg_ref)
    )


def _mm_first(x2, w2, deg2):
    return pl.pallas_call(
        _mm_first_body,
        grid=(2, _G),
        in_specs=[
            pl.BlockSpec((1, _R, _D), lambda b, i: (b, i, 0)),
            pl.BlockSpec((1, _D, _D), lambda b, i: (b, 0, 0)),
            pl.BlockSpec((1, _R, 8), lambda b, i: (b, i, 0)),
        ],
        out_specs=pl.BlockSpec((1, _R, _D), lambda b, i: (b, i, 0)),
        out_shape=jax.ShapeDtypeStruct((2, _NP, _D), jnp.float32),
    )(x2, w2, deg2)


def _mm_mid_body(acc_ref, z_ref, deg_ref, b_ref, w_ref, o_ref):
    dis = _dis(deg_ref)
    h = jnp.maximum((acc_ref[0] + z_ref[0]) * dis + b_ref[0], 0.0)
    o_ref[0] = jnp.dot(h, w_ref[0], preferred_element_type=jnp.float32) * dis


def _mm_mid(acc2, z2, deg2, b2, w2):
    return pl.pallas_call(
        _mm_mid_body,
        grid=(2, _G),
        in_specs=[
            pl.BlockSpec((1, _R, _D), lambda b, i: (b, i, 0)),
            pl.BlockSpec((1, _R, _D), lambda b, i: (b, i, 0)),
            pl.BlockSpec((1, _R, 8), lambda b, i: (b, i, 0)),
            pl.BlockSpec((1, 1, _D), lambda b, i: (b, 0, 0)),
            pl.BlockSpec((1, _D, _D), lambda b, i: (b, 0, 0)),
        ],
        out_specs=pl.BlockSpec((1, _R, _D), lambda b, i: (b, i, 0)),
        out_shape=jax.ShapeDtypeStruct((2, _NP, _D), jnp.float32),
    )(acc2, z2, deg2, b2, w2)


def _final_body(acc_ref, z_ref, deg_ref, b_ref, o_ref):
    o_ref[0] = (acc_ref[0] + z_ref[0]) * _dis(deg_ref) + b_ref[0]


def _final(acc2, z2, deg2, b2):
    return pl.pallas_call(
        _final_body,
        grid=(2, _G),
        in_specs=[
            pl.BlockSpec((1, _R, _D), lambda b, i: (b, i, 0)),
            pl.BlockSpec((1, _R, _D), lambda b, i: (b, i, 0)),
            pl.BlockSpec((1, _R, 8), lambda b, i: (b, i, 0)),
            pl.BlockSpec((1, 1, _D), lambda b, i: (b, 0, 0)),
        ],
        out_specs=pl.BlockSpec((1, _R, _D), lambda b, i: (b, i, 0)),
        out_shape=jax.ShapeDtypeStruct((2, _NP, _D), jnp.float32),
    )(acc2, z2, deg2, b2)


def _prep_edges(edge_index):
    pad = jnp.full((_EPAD - _E,), _N, dtype=jnp.int32)
    src = jnp.concatenate([edge_index[0], pad]).reshape(16, _CH, _CK)
    dst = jnp.concatenate([edge_index[1], pad]).reshape(16, _CH, _CK)
    return src, dst


def kernel(x_data_matrix, x_edge_index, y_data_matrix, y_edge_index,
           Wx1, bx1, Wx2, bx2, Wx3, bx3, Wx4, bx4,
           Wy1, by1, Wy2, by2, Wy3, by3, Wy4, by4):
    zeros_rpt = jnp.zeros((_RPT, _D), dtype=jnp.float32)
    sx, dx = _prep_edges(x_edge_index)
    sy, dy = _prep_edges(y_edge_index)
    src = jnp.stack([sx, sy])
    dst = jnp.stack([dx, dy])
    xp = jnp.pad(x_data_matrix, ((0, _NP - _N), (0, 0)))
    yp = jnp.pad(y_data_matrix, ((0, _NP - _N), (0, 0)))
    x2 = jnp.stack([xp, yp])
    w1 = jnp.stack([Wx1, Wy1])
    ws = (jnp.stack([Wx2, Wy2]), jnp.stack([Wx3, Wy3]), jnp.stack([Wx4, Wy4]))
    bs = (jnp.stack([bx1, by1]).reshape(2, 1, _D),
          jnp.stack([bx2, by2]).reshape(2, 1, _D),
          jnp.stack([bx3, by3]).reshape(2, 1, _D),
          jnp.stack([bx4, by4]).reshape(2, 1, _D))

    ones2 = jnp.ones((2, _NP, _D), dtype=jnp.float32)
    deg2 = lax.slice(_spmm_call(ones2, src, dst, zeros_rpt),
                     (0, 0, 0), (2, _NP, 8))
    z = _mm_first(x2, w1, deg2)
    for layer in range(3):
        acc = _spmm_call(z, src, dst, zeros_rpt)
        z = _mm_mid(acc, z, deg2, bs[layer], ws[layer])
    acc = _spmm_call(z, src, dst, zeros_rpt)
    out = _final(acc, z, deg2, bs[3])
    return (out[0, :_N], out[1, :_N])
